# baseline (speedup 1.0000x reference)
import functools

import jax
import jax.numpy as jnp
from jax import lax
from jax.experimental import pallas as pl
from jax.experimental.pallas import tpu as pltpu
from jax.experimental.pallas import tpu_sc as plsc

N = 10000
E = 320000
D = 128

NC = 2
NS = 16
NW = NC * NS
L = 16

CH = 128
NCHUNK = 80
EPW = NCHUNK * CH
EP = NW * EPW
NPAIR = NCHUNK // 2
FB = 80
NFL = 7
RPS = FB * NFL
NEXTRA = (N - NS * RPS) // FB

assert EP >= E and CH % 8 == 0 and NCHUNK == 2 * NPAIR
assert NS * RPS + NEXTRA * FB == N and NEXTRA <= NS


def _agg_body(x_hbm, src_hbm, dst_hbm, iota_hbm, zrow_hbm, z16_hbm, e0_hbm,
              acc_out, deg_out,
              src_v, dst_v, rows_v, ones_v,
              src_b, dst_b, rows_b, fidx, acc_sh, deg_sh, sem, sem_b):
  c = lax.axis_index("c")
  s = lax.axis_index("s")
  wid = c * NS + s

  r0 = s * RPS

  def _blk_off(j):
    extra = jnp.where(s < NEXTRA, NS * RPS + s * FB, r0)
    return jnp.where(j < NFL, r0 + j * FB, extra)

  pltpu.sync_copy(zrow_hbm, rows_v)
  pltpu.sync_copy(z16_hbm, ones_v)

  @pl.loop(0, NFL + 1)
  def _init(j):
    off = _blk_off(j)
    pltpu.sync_copy(iota_hbm.at[pl.ds(off, FB)], fidx)
    pltpu.sync_copy(rows_v.at[pl.ds(0, FB)], acc_sh.at[fidx])
    pltpu.sync_copy(ones_v.at[pl.ds(0, FB)], deg_sh.at[fidx])

  pltpu.sync_copy(e0_hbm, ones_v)

  plsc.subcore_barrier()

  base = wid * EPW

  def _load_idx(k, sv, dv):
    off = base + k * CH
    pltpu.sync_copy(src_hbm.at[pl.ds(off, CH)], sv)
    pltpu.sync_copy(dst_hbm.at[pl.ds(off, CH)], dv)

  def _scatter(rv, dv):
    pltpu.sync_copy(rv, acc_sh.at[dv], add=True)
    pltpu.sync_copy(ones_v, deg_sh.at[dv], add=True)

  _load_idx(0, src_v, dst_v)
  pltpu.make_async_copy(x_hbm.at[src_v], rows_v, sem).start()

  @pl.loop(0, NPAIR)
  def _pair(t):
    _load_idx(2 * t + 1, src_b, dst_b)
    pltpu.make_async_copy(x_hbm.at[src_b], rows_b, sem_b).start()
    pltpu.make_async_copy(x_hbm.at[src_v], rows_v, sem).wait()
    _scatter(rows_v, dst_v)
    nxt = jnp.minimum(2 * t + 2, NCHUNK - 1)
    _load_idx(nxt, src_v, dst_v)
    pltpu.make_async_copy(x_hbm.at[src_v], rows_v, sem).start()
    pltpu.make_async_copy(x_hbm.at[src_b], rows_b, sem_b).wait()
    _scatter(rows_b, dst_b)

  pltpu.make_async_copy(x_hbm.at[src_v], rows_v, sem).wait()

  plsc.subcore_barrier()

  @pl.loop(0, NFL + 1)
  def _flush(j):
    off = _blk_off(j)
    pltpu.sync_copy(iota_hbm.at[pl.ds(off, FB)], fidx)
    pltpu.async_copy(acc_sh.at[fidx], rows_v.at[pl.ds(0, FB)], sem).wait()
    pltpu.sync_copy(rows_v.at[pl.ds(0, FB)], acc_out.at[c, pl.ds(off, FB)])
    pltpu.async_copy(deg_sh.at[fidx], ones_v.at[pl.ds(0, FB)], sem).wait()
    pltpu.sync_copy(ones_v.at[pl.ds(0, FB)], deg_out.at[c, pl.ds(off, FB)])


_agg = pl.kernel(
    _agg_body,
    out_type=(
        jax.ShapeDtypeStruct((NC, N, D), jnp.float32),
        jax.ShapeDtypeStruct((NC, N, L), jnp.float32),
    ),
    mesh=plsc.VectorSubcoreMesh(
        core_axis_name="c", subcore_axis_name="s",
        num_cores=NC, num_subcores=NS),
    compiler_params=pltpu.CompilerParams(use_tc_tiling_on_sc=False),
    scratch_types=[
        pltpu.VMEM((CH,), jnp.int32),
        pltpu.VMEM((CH,), jnp.int32),
        pltpu.VMEM((CH, D), jnp.float32),
        pltpu.VMEM((CH, L), jnp.float32),
        pltpu.VMEM((CH,), jnp.int32),
        pltpu.VMEM((CH,), jnp.int32),
        pltpu.VMEM((CH, D), jnp.float32),
        pltpu.VMEM((FB,), jnp.int32),
        pltpu.VMEM_SHARED((N + 8, D), jnp.float32),
        pltpu.VMEM_SHARED((N + 8, L), jnp.float32),
        pltpu.SemaphoreType.DMA,
        pltpu.SemaphoreType.DMA,
    ],
)


def _dense_body(relu, x_ref, acca_ref, accb_ref, dega_ref, degb_ref,
                wst_ref, wnt_ref, b_ref, out_ref):
  deg = dega_ref[:, 0:1] + degb_ref[:, 0:1]
  dinv = 1.0 / jnp.maximum(deg, 1.0)
  hn = (acca_ref[...] + accb_ref[...]) * dinv
  h = (jnp.dot(x_ref[...], wst_ref[...], preferred_element_type=jnp.float32)
       + jnp.dot(hn, wnt_ref[...], preferred_element_type=jnp.float32)
       + b_ref[...])
  if relu:
    h = jnp.maximum(h, 0.0)
  out_ref[...] = h


BR = 1000


def _dense(x, acc, deg, w_self, w_neigh, b, relu):
  row_spec = pl.BlockSpec((BR, D), lambda i: (i, 0))
  deg_spec = pl.BlockSpec((BR, L), lambda i: (i, 0))
  full_spec = pl.BlockSpec((D, D), lambda i: (0, 0))
  b_spec = pl.BlockSpec((1, D), lambda i: (0, 0))
  return pl.pallas_call(
      functools.partial(_dense_body, relu),
      grid=(N // BR,),
      in_specs=[row_spec, row_spec, row_spec, deg_spec, deg_spec,
                full_spec, full_spec, b_spec],
      out_specs=row_spec,
      out_shape=jax.ShapeDtypeStruct((N, D), jnp.float32),
  )(x, acc[0], acc[1], deg[0], deg[1],
    w_self.T, w_neigh.T, b.reshape(1, D))


def kernel(feat, edge_index1, edge_index2, W_self1, W_neigh1, b1,
           W_self2, W_neigh2, b2):
  def _pad(v, fill):
    return jnp.concatenate(
        [v.astype(jnp.int32), jnp.full((EP - E,), fill, jnp.int32)])

  src1 = _pad(edge_index1[0], 0)
  dst1 = _pad(edge_index1[1], N)
  src2 = _pad(edge_index2[0], 0)
  dst2 = _pad(edge_index2[1], N)
  iota = jnp.arange(N, dtype=jnp.int32)
  zrow = jnp.zeros((CH, D), jnp.float32)
  z16 = jnp.zeros((CH, L), jnp.float32)
  e0 = jnp.zeros((CH, L), jnp.float32).at[:, 0].set(1.0)

  acc1, deg1 = _agg(feat, src1, dst1, iota, zrow, z16, e0)
  h1 = _dense(feat, acc1, deg1, W_self1, W_neigh1, b1, relu=True)
  acc2, deg2 = _agg(h1, src2, dst2, iota, zrow, z16, e0)
  return _dense(h1, acc2, deg2, W_self2, W_neigh2, b2, relu=False)


# speedup vs baseline: 1.0815x; 1.0815x over previous
import functools

import jax
import jax.numpy as jnp
from jax import lax
from jax.experimental import pallas as pl
from jax.experimental.pallas import tpu as pltpu
from jax.experimental.pallas import tpu_sc as plsc

N = 10000
E = 320000
D = 128

NC = 2
NS = 16
NW = NC * NS
L = 16

CH = 128
NCHUNK = 80
EPW = NCHUNK * CH
EP = NW * EPW
NPAIR = NCHUNK // 2
FB = 80
NFL = 7
RPS = FB * NFL
NEXTRA = (N - NS * RPS) // FB
NDUMP = 128

assert EP >= E and CH % 8 == 0 and NCHUNK == 2 * NPAIR
assert NS * RPS + NEXTRA * FB == N and NEXTRA <= NS


def _agg_body(x_hbm, src_hbm, dst_hbm, iota_hbm, zrow_hbm, z16_hbm, e0_hbm,
              acc_out, deg_out,
              src_v, dst_v, rows_v, ones_v,
              src_b, dst_b, rows_b, fidx, acc_sh, deg_sh, sem, sem_b):
  c = lax.axis_index("c")
  s = lax.axis_index("s")
  wid = c * NS + s

  r0 = s * RPS

  def _blk_off(j):
    extra = jnp.where(s < NEXTRA, NS * RPS + s * FB, r0)
    return jnp.where(j < NFL, r0 + j * FB, extra)

  pltpu.sync_copy(zrow_hbm, rows_v)
  pltpu.sync_copy(z16_hbm, ones_v)

  @pl.loop(0, NFL + 1)
  def _init(j):
    off = _blk_off(j)
    pltpu.sync_copy(iota_hbm.at[pl.ds(off, FB)], fidx)
    pltpu.sync_copy(rows_v.at[pl.ds(0, FB)], acc_sh.at[fidx])
    pltpu.sync_copy(ones_v.at[pl.ds(0, FB)], deg_sh.at[fidx])

  pltpu.sync_copy(e0_hbm, ones_v)

  plsc.subcore_barrier()

  base = wid * EPW

  def _load_idx(k, sv, dv):
    off = base + k * CH
    pltpu.sync_copy(src_hbm.at[pl.ds(off, CH)], sv)
    pltpu.sync_copy(dst_hbm.at[pl.ds(off, CH)], dv)

  def _scatter(rv, dv):
    pltpu.sync_copy(rv, acc_sh.at[dv], add=True)
    pltpu.sync_copy(ones_v, deg_sh.at[dv], add=True)

  _load_idx(0, src_v, dst_v)
  pltpu.make_async_copy(x_hbm.at[src_v], rows_v, sem).start()

  @pl.loop(0, NPAIR)
  def _pair(t):
    _load_idx(2 * t + 1, src_b, dst_b)
    pltpu.make_async_copy(x_hbm.at[src_b], rows_b, sem_b).start()
    pltpu.make_async_copy(x_hbm.at[src_v], rows_v, sem).wait()
    _scatter(rows_v, dst_v)
    nxt = jnp.minimum(2 * t + 2, NCHUNK - 1)
    _load_idx(nxt, src_v, dst_v)
    pltpu.make_async_copy(x_hbm.at[src_v], rows_v, sem).start()
    pltpu.make_async_copy(x_hbm.at[src_b], rows_b, sem_b).wait()
    _scatter(rows_b, dst_b)

  pltpu.make_async_copy(x_hbm.at[src_v], rows_v, sem).wait()

  plsc.subcore_barrier()

  @pl.loop(0, NFL + 1)
  def _flush(j):
    off = _blk_off(j)
    pltpu.sync_copy(iota_hbm.at[pl.ds(off, FB)], fidx)
    pltpu.async_copy(acc_sh.at[fidx], rows_v.at[pl.ds(0, FB)], sem).wait()
    pltpu.sync_copy(rows_v.at[pl.ds(0, FB)], acc_out.at[c, pl.ds(off, FB)])
    pltpu.async_copy(deg_sh.at[fidx], ones_v.at[pl.ds(0, FB)], sem).wait()
    pltpu.sync_copy(ones_v.at[pl.ds(0, FB)], deg_out.at[c, pl.ds(off, FB)])


_agg = pl.kernel(
    _agg_body,
    out_type=(
        jax.ShapeDtypeStruct((NC, N, D), jnp.float32),
        jax.ShapeDtypeStruct((NC, N, L), jnp.float32),
    ),
    mesh=plsc.VectorSubcoreMesh(
        core_axis_name="c", subcore_axis_name="s",
        num_cores=NC, num_subcores=NS),
    compiler_params=pltpu.CompilerParams(use_tc_tiling_on_sc=False),
    scratch_types=[
        pltpu.VMEM((CH,), jnp.int32),
        pltpu.VMEM((CH,), jnp.int32),
        pltpu.VMEM((CH, D), jnp.float32),
        pltpu.VMEM((CH, L), jnp.float32),
        pltpu.VMEM((CH,), jnp.int32),
        pltpu.VMEM((CH,), jnp.int32),
        pltpu.VMEM((CH, D), jnp.float32),
        pltpu.VMEM((FB,), jnp.int32),
        pltpu.VMEM_SHARED((N + NDUMP, D), jnp.float32),
        pltpu.VMEM_SHARED((N + NDUMP, L), jnp.float32),
        pltpu.SemaphoreType.DMA,
        pltpu.SemaphoreType.DMA,
    ],
)


def _dense_body(relu, x_ref, acca_ref, accb_ref, dega_ref, degb_ref,
                wst_ref, wnt_ref, b_ref, out_ref):
  deg = dega_ref[:, 0:1] + degb_ref[:, 0:1]
  dinv = 1.0 / jnp.maximum(deg, 1.0)
  hn = (acca_ref[...] + accb_ref[...]) * dinv
  h = (jnp.dot(x_ref[...], wst_ref[...], preferred_element_type=jnp.float32)
       + jnp.dot(hn, wnt_ref[...], preferred_element_type=jnp.float32)
       + b_ref[...])
  if relu:
    h = jnp.maximum(h, 0.0)
  out_ref[...] = h


BR = 1000


def _dense(x, acc, deg, w_self, w_neigh, b, relu):
  row_spec = pl.BlockSpec((BR, D), lambda i: (i, 0))
  deg_spec = pl.BlockSpec((BR, L), lambda i: (i, 0))
  full_spec = pl.BlockSpec((D, D), lambda i: (0, 0))
  b_spec = pl.BlockSpec((1, D), lambda i: (0, 0))
  return pl.pallas_call(
      functools.partial(_dense_body, relu),
      grid=(N // BR,),
      in_specs=[row_spec, row_spec, row_spec, deg_spec, deg_spec,
                full_spec, full_spec, b_spec],
      out_specs=row_spec,
      out_shape=jax.ShapeDtypeStruct((N, D), jnp.float32),
  )(x, acc[0], acc[1], deg[0], deg[1],
    w_self.T, w_neigh.T, b.reshape(1, D))


def kernel(feat, edge_index1, edge_index2, W_self1, W_neigh1, b1,
           W_self2, W_neigh2, b2):
  dump = N + (jnp.arange(EP - E, dtype=jnp.int32) % NDUMP)

  def _pad(v, fill):
    return jnp.concatenate([v.astype(jnp.int32), fill])

  zpad = jnp.zeros((EP - E,), jnp.int32)
  src1 = _pad(edge_index1[0], zpad)
  dst1 = _pad(edge_index1[1], dump)
  src2 = _pad(edge_index2[0], zpad)
  dst2 = _pad(edge_index2[1], dump)
  iota = jnp.arange(N, dtype=jnp.int32)
  zrow = jnp.zeros((CH, D), jnp.float32)
  z16 = jnp.zeros((CH, L), jnp.float32)
  e0 = jnp.zeros((CH, L), jnp.float32).at[:, 0].set(1.0)

  acc1, deg1 = _agg(feat, src1, dst1, iota, zrow, z16, e0)
  h1 = _dense(feat, acc1, deg1, W_self1, W_neigh1, b1, relu=True)
  acc2, deg2 = _agg(h1, src2, dst2, iota, zrow, z16, e0)
  return _dense(h1, acc2, deg2, W_self2, W_neigh2, b2, relu=False)


# speedup vs baseline: 2.2059x; 2.0397x over previous
import functools

import jax
import jax.numpy as jnp
from jax import lax
from jax.experimental import pallas as pl
from jax.experimental.pallas import tpu as pltpu
from jax.experimental.pallas import tpu_sc as plsc

N = 10000
E = 320000
D = 128

NC = 2
NS = 16
NW = NC * NS
L = 16

EPW = E // NW
CH = 80
NCHUNK = EPW // CH
NPAIR = (NCHUNK - 1) // 2
NFL = 7
RPS = CH * NFL
NEXTRA = (N - NS * RPS) // CH

assert EPW * NW == E and NCHUNK * CH == EPW and CH % 8 == 0
assert NCHUNK == 2 * NPAIR + 1
assert NS * RPS + NEXTRA * CH == N and NEXTRA <= NS


def _agg_body(x_hbm, src_hbm, dst_hbm, iota_hbm, zrow_hbm, z16_hbm, e0_hbm,
              acc_out, deg_out,
              src_v, dst_v, rows_v, ones_v,
              src_b, dst_b, rows_b, acc_sh, deg_sh, sem, sem_b):
  c = lax.axis_index("c")
  s = lax.axis_index("s")
  wid = c * NS + s

  r0 = s * RPS

  def _blk_off(j):
    extra = jnp.where(s < NEXTRA, NS * RPS + s * CH, r0)
    return jnp.where(j < NFL, r0 + j * CH, extra)

  pltpu.sync_copy(zrow_hbm, rows_v)
  pltpu.sync_copy(z16_hbm, ones_v)

  @pl.loop(0, NFL + 1)
  def _init(j):
    off = _blk_off(j)
    pltpu.sync_copy(iota_hbm.at[pl.ds(off, CH)], src_v)
    pltpu.sync_copy(rows_v, acc_sh.at[src_v])
    pltpu.sync_copy(ones_v, deg_sh.at[src_v])

  pltpu.sync_copy(e0_hbm, ones_v)

  plsc.subcore_barrier()

  base = wid * EPW

  def _load_idx(k, sv, dv):
    off = base + k * CH
    pltpu.sync_copy(src_hbm.at[pl.ds(off, CH)], sv)
    pltpu.sync_copy(dst_hbm.at[pl.ds(off, CH)], dv)

  def _scatter(rv, dv):
    pltpu.sync_copy(rv, acc_sh.at[dv], add=True)
    pltpu.sync_copy(ones_v, deg_sh.at[dv], add=True)

  _load_idx(0, src_v, dst_v)
  pltpu.make_async_copy(x_hbm.at[src_v], rows_v, sem).start()

  @pl.loop(0, NPAIR)
  def _pair(t):
    _load_idx(2 * t + 1, src_b, dst_b)
    pltpu.make_async_copy(x_hbm.at[src_b], rows_b, sem_b).start()
    pltpu.make_async_copy(x_hbm.at[src_v], rows_v, sem).wait()
    _scatter(rows_v, dst_v)
    _load_idx(2 * t + 2, src_v, dst_v)
    pltpu.make_async_copy(x_hbm.at[src_v], rows_v, sem).start()
    pltpu.make_async_copy(x_hbm.at[src_b], rows_b, sem_b).wait()
    _scatter(rows_b, dst_b)

  pltpu.make_async_copy(x_hbm.at[src_v], rows_v, sem).wait()
  _scatter(rows_v, dst_v)

  plsc.subcore_barrier()

  @pl.loop(0, NFL + 1)
  def _flush(j):
    off = _blk_off(j)
    pltpu.sync_copy(iota_hbm.at[pl.ds(off, CH)], src_v)
    pltpu.async_copy(acc_sh.at[src_v], rows_v, sem).wait()
    pltpu.sync_copy(rows_v, acc_out.at[c, pl.ds(off, CH)])
    pltpu.async_copy(deg_sh.at[src_v], ones_v, sem).wait()
    pltpu.sync_copy(ones_v, deg_out.at[c, pl.ds(off, CH)])


_agg = pl.kernel(
    _agg_body,
    out_type=(
        jax.ShapeDtypeStruct((NC, N, D), jnp.float32),
        jax.ShapeDtypeStruct((NC, N, L), jnp.float32),
    ),
    mesh=plsc.VectorSubcoreMesh(
        core_axis_name="c", subcore_axis_name="s",
        num_cores=NC, num_subcores=NS),
    compiler_params=pltpu.CompilerParams(use_tc_tiling_on_sc=False),
    scratch_types=[
        pltpu.VMEM((CH,), jnp.int32),
        pltpu.VMEM((CH,), jnp.int32),
        pltpu.VMEM((CH, D), jnp.float32),
        pltpu.VMEM((CH, L), jnp.float32),
        pltpu.VMEM((CH,), jnp.int32),
        pltpu.VMEM((CH,), jnp.int32),
        pltpu.VMEM((CH, D), jnp.float32),
        pltpu.VMEM_SHARED((N, D), jnp.float32),
        pltpu.VMEM_SHARED((N, L), jnp.float32),
        pltpu.SemaphoreType.DMA,
        pltpu.SemaphoreType.DMA,
    ],
)


def _dense_body(relu, x_ref, acca_ref, accb_ref, dega_ref, degb_ref,
                wst_ref, wnt_ref, b_ref, out_ref):
  deg = dega_ref[:, 0:1] + degb_ref[:, 0:1]
  dinv = 1.0 / jnp.maximum(deg, 1.0)
  hn = (acca_ref[...] + accb_ref[...]) * dinv
  h = (jnp.dot(x_ref[...], wst_ref[...], preferred_element_type=jnp.float32)
       + jnp.dot(hn, wnt_ref[...], preferred_element_type=jnp.float32)
       + b_ref[...])
  if relu:
    h = jnp.maximum(h, 0.0)
  out_ref[...] = h


BR = 1000


def _dense(x, acc, deg, w_self, w_neigh, b, relu):
  row_spec = pl.BlockSpec((BR, D), lambda i: (i, 0))
  deg_spec = pl.BlockSpec((BR, L), lambda i: (i, 0))
  full_spec = pl.BlockSpec((D, D), lambda i: (0, 0))
  b_spec = pl.BlockSpec((1, D), lambda i: (0, 0))
  return pl.pallas_call(
      functools.partial(_dense_body, relu),
      grid=(N // BR,),
      in_specs=[row_spec, row_spec, row_spec, deg_spec, deg_spec,
                full_spec, full_spec, b_spec],
      out_specs=row_spec,
      out_shape=jax.ShapeDtypeStruct((N, D), jnp.float32),
  )(x, acc[0], acc[1], deg[0], deg[1],
    w_self.T, w_neigh.T, b.reshape(1, D))


def kernel(feat, edge_index1, edge_index2, W_self1, W_neigh1, b1,
           W_self2, W_neigh2, b2):
  src1 = edge_index1[0].astype(jnp.int32)
  dst1 = edge_index1[1].astype(jnp.int32)
  src2 = edge_index2[0].astype(jnp.int32)
  dst2 = edge_index2[1].astype(jnp.int32)
  iota = jnp.arange(N, dtype=jnp.int32)
  zrow = jnp.zeros((CH, D), jnp.float32)
  z16 = jnp.zeros((CH, L), jnp.float32)
  e0 = jnp.zeros((CH, L), jnp.float32).at[:, 0].set(1.0)

  acc1, deg1 = _agg(feat, src1, dst1, iota, zrow, z16, e0)
  h1 = _dense(feat, acc1, deg1, W_self1, W_neigh1, b1, relu=True)
  acc2, deg2 = _agg(h1, src2, dst2, iota, zrow, z16, e0)
  return _dense(h1, acc2, deg2, W_self2, W_neigh2, b2, relu=False)
